# baseline (speedup 1.0000x reference)
import functools

import jax
import jax.numpy as jnp
from jax import lax
from jax.experimental import pallas as pl
from jax.experimental.pallas import tpu as pltpu
from jax.experimental.pallas import tpu_sc as plsc

_SPK_NUM = 1000000
_EMB_DIM = 64
_BATCH = 16384

_NC = 2
_NS = 16
_L = 16
_NW = _NC * _NS
_B_PER_W = _BATCH // _NW


def _rsqrt(x):
    i = lax.bitcast_convert_type(x, jnp.int32)
    i = jnp.int32(0x5F3759DF) - lax.shift_right_arithmetic(i, 1)
    y = lax.bitcast_convert_type(i, jnp.float32)
    for _ in range(3):
        y = y * (1.5 - 0.5 * x * y * y)
    return y


_mesh = plsc.VectorSubcoreMesh(core_axis_name="c", subcore_axis_name="s")


@functools.partial(
    pl.kernel,
    out_type=jax.ShapeDtypeStruct((_BATCH, _EMB_DIM), jnp.float32),
    mesh=_mesh,
    scratch_types=[
        pltpu.VMEM((_B_PER_W,), jnp.int32),
        pltpu.VMEM((_B_PER_W, _EMB_DIM), jnp.float32),
        pltpu.SemaphoreType.DMA,
    ],
    compiler_params=pltpu.CompilerParams(
        needs_layout_passes=False, use_tc_tiling_on_sc=False
    ),
)
def _embed_normalize(idx_hbm, table_hbm, out_hbm, idx_v, rows_v, sem):
    wid = lax.axis_index("s") * _NC + lax.axis_index("c")
    base = wid * _B_PER_W

    pltpu.sync_copy(idx_hbm.at[pl.ds(base, _B_PER_W)], idx_v)
    pltpu.async_copy(table_hbm.at[idx_v], rows_v, sem).wait()

    lane = lax.iota(jnp.int32, _L)

    def norm_block(b, _):
        rows = b * _L + lane

        def ssq_col(c, acc):
            col = jnp.full((_L,), c, jnp.int32)
            v = plsc.load_gather(rows_v, [rows, col])
            return acc + v * v

        ssq = lax.fori_loop(0, _EMB_DIM, ssq_col, jnp.zeros((_L,), jnp.float32))
        inv = jnp.where(ssq > 1e-24, _rsqrt(ssq), 1e12)

        def scale_col(c, _):
            col = jnp.full((_L,), c, jnp.int32)
            v = plsc.load_gather(rows_v, [rows, col])
            plsc.store_scatter(rows_v, [rows, col], v * inv)
            return 0

        lax.fori_loop(0, _EMB_DIM, scale_col, 0)
        return 0

    lax.fori_loop(0, _B_PER_W // _L, norm_block, 0)

    pltpu.sync_copy(rows_v, out_hbm.at[pl.ds(base, _B_PER_W)])


def kernel(spk_ids, table):
    return _embed_normalize(spk_ids, table)


# speedup vs baseline: 1.6476x; 1.6476x over previous
import functools

import jax
import jax.numpy as jnp
from jax import lax
from jax.experimental import pallas as pl
from jax.experimental.pallas import tpu as pltpu
from jax.experimental.pallas import tpu_sc as plsc

_SPK_NUM = 1000000
_EMB_DIM = 64
_BATCH = 16384

_NC = 2
_NS = 16
_L = 16
_NW = _NC * _NS
_B_PER_W = _BATCH // _NW


def _rsqrt(x):
    i = lax.bitcast_convert_type(x, jnp.int32)
    i = jnp.int32(0x5F3759DF) - lax.shift_right_arithmetic(i, 1)
    y = lax.bitcast_convert_type(i, jnp.float32)
    for _ in range(3):
        y = y * (1.5 - 0.5 * x * y * y)
    return y


_mesh = plsc.VectorSubcoreMesh(core_axis_name="c", subcore_axis_name="s")


@functools.partial(
    pl.kernel,
    out_type=jax.ShapeDtypeStruct((_BATCH, _EMB_DIM), jnp.float32),
    mesh=_mesh,
    scratch_types=[
        pltpu.VMEM((_B_PER_W,), jnp.int32),
        pltpu.VMEM((_B_PER_W, _EMB_DIM), jnp.float32),
        pltpu.SemaphoreType.DMA,
    ],
    compiler_params=pltpu.CompilerParams(needs_layout_passes=False),
)
def _embed_normalize(idx_hbm, table_hbm, out_hbm, idx_v, rows_v, sem):
    wid = lax.axis_index("s") * _NC + lax.axis_index("c")
    base = wid * _B_PER_W

    pltpu.sync_copy(idx_hbm.at[pl.ds(base, _B_PER_W)], idx_v)

    def fire(g, _):
        vec = idx_v[pl.ds(g * _L, _L)]
        for j in range(_L):
            row = vec[j]
            pltpu.async_copy(
                table_hbm.at[pl.ds(row, 1)], rows_v.at[pl.ds(g * _L + j, 1)], sem
            )
        return 0

    lax.fori_loop(0, _B_PER_W // _L, fire, 0)

    def drain(i, _):
        pltpu.make_async_copy(
            table_hbm.at[pl.ds(0, 1)], rows_v.at[pl.ds(i, 1)], sem
        ).wait()
        return 0

    lax.fori_loop(0, _B_PER_W, drain, 0)

    lane = lax.iota(jnp.int32, _L)

    def norm_block(b, _):
        rows = b * _L + lane

        acc = jnp.zeros((_L,), jnp.float32)
        for c in range(_EMB_DIM):
            col = jnp.full((_L,), c, jnp.int32)
            v = plsc.load_gather(rows_v, [rows, col])
            acc = acc + v * v

        inv = jnp.where(acc > 1e-24, _rsqrt(acc), 1e12)

        for c in range(_EMB_DIM):
            col = jnp.full((_L,), c, jnp.int32)
            v = plsc.load_gather(rows_v, [rows, col])
            plsc.store_scatter(rows_v, [rows, col], v * inv)
        return 0

    lax.fori_loop(0, _B_PER_W // _L, norm_block, 0)

    pltpu.sync_copy(rows_v, out_hbm.at[pl.ds(base, _B_PER_W)])


def kernel(spk_ids, table):
    return _embed_normalize(spk_ids, table)


# speedup vs baseline: 3.2614x; 1.9795x over previous
import functools

import jax
import jax.numpy as jnp
from jax import lax
from jax.experimental import pallas as pl
from jax.experimental.pallas import tpu as pltpu
from jax.experimental.pallas import tpu_sc as plsc

_SPK_NUM = 1000000
_EMB_DIM = 64
_BATCH = 16384

_NC = 2
_NS = 16
_L = 16
_NW = _NC * _NS
_B_PER_W = _BATCH // _NW
_NG = _B_PER_W // _L
_K = 8


def _rsqrt(x):
    i = lax.bitcast_convert_type(x, jnp.int32)
    i = jnp.int32(0x5F3759DF) - lax.shift_right_arithmetic(i, 1)
    y = lax.bitcast_convert_type(i, jnp.float32)
    for _ in range(3):
        y = y * (1.5 - 0.5 * x * y * y)
    return y


_mesh = plsc.VectorSubcoreMesh(core_axis_name="c", subcore_axis_name="s")


@functools.partial(
    pl.kernel,
    out_type=jax.ShapeDtypeStruct((_EMB_DIM, _BATCH), jnp.float32),
    mesh=_mesh,
    scratch_types=[
        pltpu.VMEM((_B_PER_W,), jnp.int32),
        pltpu.VMEM((_K, _EMB_DIM, 128), jnp.float32),
        pltpu.VMEM((_EMB_DIM, _B_PER_W), jnp.float32),
        pltpu.SemaphoreType.DMA((_K,)),
    ],
    compiler_params=pltpu.CompilerParams(needs_layout_passes=False),
)
def _embed_normalize(idx_hbm, tableT_hbm, outT_hbm, idx_v, ring_v, cols_v, sems):
    wid = lax.axis_index("s") * _NC + lax.axis_index("c")
    base = wid * _B_PER_W

    pltpu.sync_copy(idx_hbm.at[pl.ds(base, _B_PER_W)], idx_v)

    lane = lax.iota(jnp.int32, _L)

    def fetch(i_id, slot):
        blk = lax.shift_right_logical(i_id, 7) * 128
        pltpu.async_copy(
            tableT_hbm.at[:, pl.ds(blk, 128)], ring_v.at[slot], sems.at[slot]
        )

    vec0 = idx_v[pl.ds(0, _L)]
    for j in range(_K):
        fetch(vec0[j], j)

    def group(g, _):
        vec = idx_v[pl.ds(g * _L, _L)]
        nxt_base = jnp.minimum((g + 1) * _L, _B_PER_W - _L)
        nxt = idx_v[pl.ds(nxt_base, _L)]

        for j in range(_L):
            i = g * _L + j
            slot = j % _K
            pltpu.make_async_copy(
                tableT_hbm.at[:, pl.ds(0, 128)], ring_v.at[slot], sems.at[slot]
            ).wait()

            m = jnp.bitwise_and(vec[j], 127)
            mcol = jnp.full((_L,), m, jnp.int32)
            icol = jnp.full((_L,), i, jnp.int32)
            for k in range(_EMB_DIM // _L):
                crow = lane + k * _L
                v = plsc.load_gather(ring_v.at[slot], [crow, mcol])
                plsc.store_scatter(cols_v, [crow, icol], v)

            if j < _L - _K:
                fetch(vec[j + _K], slot)
            else:
                @pl.when(g < _NG - 1)
                def _():
                    fetch(nxt[j + _K - _L], slot)
        return 0

    lax.fori_loop(0, _NG, group, 0)

    def norm_block(g, _):
        s = g * _L

        acc = jnp.zeros((_L,), jnp.float32)
        for c in range(_EMB_DIM):
            v = cols_v[c, pl.ds(s, _L)]
            acc = acc + v * v

        inv = jnp.where(acc > 1e-24, _rsqrt(acc), 1e12)

        for c in range(_EMB_DIM):
            cols_v[c, pl.ds(s, _L)] = cols_v[c, pl.ds(s, _L)] * inv
        return 0

    lax.fori_loop(0, _NG, norm_block, 0)

    pltpu.sync_copy(cols_v, outT_hbm.at[:, pl.ds(base, _B_PER_W)])


def kernel(spk_ids, table):
    outT = _embed_normalize(spk_ids, table.T)
    return outT.T
